# baseline (speedup 1.0000x reference)
import functools

import numpy as np
import jax
import jax.numpy as jnp
from jax import lax
from jax.experimental import pallas as pl
from jax.experimental.pallas import tpu as pltpu
from jax.experimental.pallas import tpu_sc as plsc

B = 256
N = 512
OUT_LEN = N * (N + 1) // 2
GROUP_ROWS = 32
NGROUPS = N // GROUP_ROWS
SLAB = GROUP_ROWS * N

NC, NS = 2, 16
NW = NC * NS
BATCH_PER_W = B // NW

_r, _c = np.triu_indices(N)
_flat = (_r * N + _c).astype(np.int64)
_g = _r // GROUP_ROWS
IDX_LOCAL = (_flat - _g * SLAB).astype(np.int32)

GLEN = [int(np.sum(_g == g)) for g in range(NGROUPS)]
GOFF = [int(np.searchsorted(_g, g)) for g in range(NGROUPS)]
LMAX = GLEN[0]


@functools.partial(
    pl.kernel,
    mesh=plsc.VectorSubcoreMesh(core_axis_name="c", subcore_axis_name="s"),
    out_type=jax.ShapeDtypeStruct((B * OUT_LEN,), jnp.float32),
    compiler_params=pltpu.CompilerParams(needs_layout_passes=False),
    scratch_types=[
        pltpu.VMEM((SLAB,), jnp.float32),
        pltpu.VMEM((LMAX,), jnp.float32),
        pltpu.VMEM((LMAX,), jnp.int32),
    ],
)
def _triu_sc(x_ref, idx_ref, out_ref, slab_v, buf_v, idxg_v):
    wid = lax.axis_index("s") * NC + lax.axis_index("c")
    for g in range(NGROUPS):
        goff, glen = GOFF[g], GLEN[g]
        nch = glen // 16
        pltpu.sync_copy(idx_ref.at[pl.ds(goff, glen)], idxg_v.at[pl.ds(0, glen)])

        def batch_body(i, _, goff=goff, glen=glen, nch=nch, g=g):
            batch = wid * BATCH_PER_W + i
            pltpu.sync_copy(
                x_ref.at[pl.ds(batch * (N * N) + g * SLAB, SLAB)], slab_v)

            def chunk_body(q, _):
                ids = idxg_v[pl.ds(q * 16, 16)]
                buf_v[pl.ds(q * 16, 16)] = plsc.load_gather(slab_v, [ids])
                return 0

            lax.fori_loop(0, nch, chunk_body, 0)
            pltpu.sync_copy(
                buf_v.at[pl.ds(0, glen)],
                out_ref.at[pl.ds(batch * OUT_LEN + goff, glen)])
            return 0

        lax.fori_loop(0, BATCH_PER_W, batch_body, 0)


def kernel(x):
    out = _triu_sc(x.reshape(-1), jnp.asarray(IDX_LOCAL))
    return out.reshape(B, OUT_LEN)


# speedup vs baseline: 1.3939x; 1.3939x over previous
import functools

import numpy as np
import jax
import jax.numpy as jnp
from jax import lax
from jax.experimental import pallas as pl
from jax.experimental.pallas import tpu as pltpu
from jax.experimental.pallas import tpu_sc as plsc

B = 256
N = 512
OUT_LEN = N * (N + 1) // 2
GROUP_ROWS = 32
NGROUPS = N // GROUP_ROWS
SLAB = GROUP_ROWS * N

NC, NS = 2, 16
NW = NC * NS
BATCH_PER_W = B // NW

_r, _c = np.triu_indices(N)
_flat = (_r * N + _c).astype(np.int64)
_g = _r // GROUP_ROWS

GLEN = [int(np.sum(_g == g)) for g in range(NGROUPS)]
GOFF = [int(np.searchsorted(_g, g)) for g in range(NGROUPS)]
PLEN = [-(-l // 64) * 64 for l in GLEN]
POFF = list(np.cumsum([0] + PLEN[:-1]))
LMAX = PLEN[0]

_idx_parts = []
for _gg in range(NGROUPS):
    _part = (_flat[GOFF[_gg]:GOFF[_gg] + GLEN[_gg]] - _gg * SLAB)
    _idx_parts.append(np.pad(_part, (0, PLEN[_gg] - GLEN[_gg])))
IDX_LOCAL = np.concatenate(_idx_parts).astype(np.int32)


@functools.partial(
    pl.kernel,
    mesh=plsc.VectorSubcoreMesh(core_axis_name="c", subcore_axis_name="s"),
    out_type=jax.ShapeDtypeStruct((B * OUT_LEN,), jnp.float32),
    compiler_params=pltpu.CompilerParams(needs_layout_passes=False),
    scratch_types=[
        pltpu.VMEM((SLAB,), jnp.float32),
        pltpu.VMEM((SLAB,), jnp.float32),
        pltpu.VMEM((LMAX,), jnp.float32),
        pltpu.VMEM((LMAX,), jnp.float32),
        pltpu.VMEM((LMAX,), jnp.int32),
        pltpu.SemaphoreType.DMA,
        pltpu.SemaphoreType.DMA,
        pltpu.SemaphoreType.DMA,
        pltpu.SemaphoreType.DMA,
    ],
)
def _triu_sc(x_ref, idx_ref, out_ref, slab0, slab1, buf0, buf1, idxg,
             ss0, ss1, ts0, ts1):
    wid = lax.axis_index("s") * NC + lax.axis_index("c")
    base = wid * BATCH_PER_W

    for g in range(NGROUPS):
        goff, glen, poff, plen = GOFF[g], GLEN[g], POFF[g], PLEN[g]
        gin = g * SLAB

        def slab_cp(b, sl, sem, gin=gin):
            return pltpu.make_async_copy(
                x_ref.at[pl.ds(b * (N * N) + gin, SLAB)], sl, sem)

        def out_cp(b, buf, sem, goff=goff, glen=glen):
            return pltpu.make_async_copy(
                buf.at[pl.ds(0, glen)],
                out_ref.at[pl.ds(b * OUT_LEN + goff, glen)], sem)

        def gather(slab, buf, plen=plen):
            def body4(q, _):
                for k in range(4):
                    o = q * 64 + k * 16
                    ids = idxg[pl.ds(o, 16)]
                    buf[pl.ds(o, 16)] = plsc.load_gather(slab, [ids])
                return 0
            lax.fori_loop(0, plen // 64, body4, 0, unroll=False)

        pltpu.sync_copy(idx_ref.at[pl.ds(poff, plen)], idxg.at[pl.ds(0, plen)])
        slab_cp(base, slab0, ss0).start()

        def pair_body(i2, _):
            i = i2 * 2
            ba = base + i
            slab_cp(ba, slab0, ss0).wait()
            slab_cp(ba + 1, slab1, ss1).start()

            @pl.when(i > 0)
            def _():
                out_cp(ba - 2, buf0, ts0).wait()

            gather(slab0, buf0)
            out_cp(ba, buf0, ts0).start()

            slab_cp(ba + 1, slab1, ss1).wait()

            @pl.when(i < BATCH_PER_W - 2)
            def _():
                slab_cp(ba + 2, slab0, ss0).start()

            @pl.when(i > 0)
            def _():
                out_cp(ba - 1, buf1, ts1).wait()

            gather(slab1, buf1)
            out_cp(ba + 1, buf1, ts1).start()
            return 0

        lax.fori_loop(0, BATCH_PER_W // 2, pair_body, 0, unroll=False)

        out_cp(base + BATCH_PER_W - 2, buf0, ts0).wait()
        out_cp(base + BATCH_PER_W - 1, buf1, ts1).wait()


def kernel(x):
    out = _triu_sc(x.reshape(-1), jnp.asarray(IDX_LOCAL))
    return out.reshape(B, OUT_LEN)


# speedup vs baseline: 1.6744x; 1.2013x over previous
import functools

import numpy as np
import jax
import jax.numpy as jnp
from jax import lax
from jax.experimental import pallas as pl
from jax.experimental.pallas import tpu as pltpu
from jax.experimental.pallas import tpu_sc as plsc

B = 256
N = 512
OUT_LEN = N * (N + 1) // 2
GROUP_ROWS = 32
NGROUPS = N // GROUP_ROWS
SLAB = GROUP_ROWS * N

NC, NS = 2, 16
NW = NC * NS
BATCH_PER_W = B // NW

_r, _c = np.triu_indices(N)
_flat = (_r * N + _c).astype(np.int64)
_g = _r // GROUP_ROWS

GLEN = [int(np.sum(_g == g)) for g in range(NGROUPS)]
GOFF = [int(np.searchsorted(_g, g)) for g in range(NGROUPS)]
PLEN = [-(-l // 64) * 64 for l in GLEN]
POFF = list(np.cumsum([0] + PLEN[:-1]))
LMAX = PLEN[0]

_idx_parts = []
for _gg in range(NGROUPS):
    _part = (_flat[GOFF[_gg]:GOFF[_gg] + GLEN[_gg]] - _gg * SLAB)
    _idx_parts.append(np.pad(_part, (0, PLEN[_gg] - GLEN[_gg])))
IDX_LOCAL = np.concatenate(_idx_parts).astype(np.int32)


@functools.partial(
    pl.kernel,
    mesh=plsc.VectorSubcoreMesh(core_axis_name="c", subcore_axis_name="s"),
    out_type=jax.ShapeDtypeStruct((B * OUT_LEN,), jnp.float32),
    compiler_params=pltpu.CompilerParams(needs_layout_passes=False),
    scratch_types=[
        pltpu.VMEM((SLAB,), jnp.float32),
        pltpu.VMEM((SLAB,), jnp.float32),
        pltpu.VMEM((LMAX,), jnp.float32),
        pltpu.VMEM((LMAX,), jnp.float32),
        pltpu.VMEM((LMAX,), jnp.int32),
        pltpu.SemaphoreType.DMA,
        pltpu.SemaphoreType.DMA,
        pltpu.SemaphoreType.DMA,
        pltpu.SemaphoreType.DMA,
    ],
)
def _triu_sc(x_ref, idx_ref, out_ref, slab0, slab1, buf0, buf1, idxg,
             ss0, ss1, ts0, ts1):
    wid = lax.axis_index("s") * NC + lax.axis_index("c")
    base = wid * BATCH_PER_W

    for g in range(NGROUPS):
        goff, glen, poff, plen = GOFF[g], GLEN[g], POFF[g], PLEN[g]
        gin = g * SLAB

        def slab_cp(b, sl, sem, gin=gin):
            return pltpu.make_async_copy(
                x_ref.at[pl.ds(b * (N * N) + gin, SLAB)], sl, sem)

        def out_cp(b, buf, sem, goff=goff, glen=glen):
            return pltpu.make_async_copy(
                buf.at[pl.ds(0, glen)],
                out_ref.at[pl.ds(b * OUT_LEN + goff, glen)], sem)

        def gather(slab, buf, plen=plen):
            @plsc.parallel_loop(0, plen, 64, unroll=4)
            def _(o):
                for k in range(4):
                    ids = idxg[pl.ds(o + k * 16, 16)]
                    buf[pl.ds(o + k * 16, 16)] = plsc.load_gather(slab, [ids])

        pltpu.sync_copy(idx_ref.at[pl.ds(poff, plen)], idxg.at[pl.ds(0, plen)])
        slab_cp(base, slab0, ss0).start()

        def pair_body(i2, _):
            i = i2 * 2
            ba = base + i
            slab_cp(ba, slab0, ss0).wait()
            slab_cp(ba + 1, slab1, ss1).start()

            @pl.when(i > 0)
            def _():
                out_cp(ba - 2, buf0, ts0).wait()

            gather(slab0, buf0)
            out_cp(ba, buf0, ts0).start()

            slab_cp(ba + 1, slab1, ss1).wait()

            @pl.when(i < BATCH_PER_W - 2)
            def _():
                slab_cp(ba + 2, slab0, ss0).start()

            @pl.when(i > 0)
            def _():
                out_cp(ba - 1, buf1, ts1).wait()

            gather(slab1, buf1)
            out_cp(ba + 1, buf1, ts1).start()
            return 0

        lax.fori_loop(0, BATCH_PER_W // 2, pair_body, 0, unroll=False)

        out_cp(base + BATCH_PER_W - 2, buf0, ts0).wait()
        out_cp(base + BATCH_PER_W - 1, buf1, ts1).wait()


def kernel(x):
    out = _triu_sc(x.reshape(-1), jnp.asarray(IDX_LOCAL))
    return out.reshape(B, OUT_LEN)


# speedup vs baseline: 1.8276x; 1.0915x over previous
import functools

import numpy as np
import jax
import jax.numpy as jnp
from jax import lax
from jax.experimental import pallas as pl
from jax.experimental.pallas import tpu as pltpu
from jax.experimental.pallas import tpu_sc as plsc

B = 256
N = 512
OUT_LEN = N * (N + 1) // 2
GROUP_ROWS = 32
NGROUPS = N // GROUP_ROWS

NC, NS = 2, 16
NW = NC * NS
BATCH_PER_W = B // NW

_r, _c = np.triu_indices(N)
_g = _r // GROUP_ROWS

GLEN = [int(np.sum(_g == g)) for g in range(NGROUPS)]
GOFF = [int(np.searchsorted(_g, g)) for g in range(NGROUPS)]
PLEN = [-(-l // 64) * 64 for l in GLEN]
POFF = list(np.cumsum([0] + PLEN[:-1]))
LMAX = PLEN[0]
GWID = [N - GROUP_ROWS * g for g in range(NGROUPS)]

_local = (_r - _g * GROUP_ROWS) * N + (_c - _g * GROUP_ROWS)
_idx_parts = []
for _gg in range(NGROUPS):
    _part = _local[GOFF[_gg]:GOFF[_gg] + GLEN[_gg]]
    _idx_parts.append(np.pad(_part, (0, PLEN[_gg] - GLEN[_gg])))
IDX_LOCAL = np.concatenate(_idx_parts).astype(np.int32)


@functools.partial(
    pl.kernel,
    mesh=plsc.VectorSubcoreMesh(core_axis_name="c", subcore_axis_name="s"),
    out_type=jax.ShapeDtypeStruct((B * OUT_LEN,), jnp.float32),
    compiler_params=pltpu.CompilerParams(
        needs_layout_passes=False, use_tc_tiling_on_sc=False),
    scratch_types=[
        pltpu.VMEM((GROUP_ROWS, N), jnp.float32),
        pltpu.VMEM((GROUP_ROWS, N), jnp.float32),
        pltpu.VMEM((LMAX,), jnp.float32),
        pltpu.VMEM((LMAX,), jnp.float32),
        pltpu.VMEM((LMAX,), jnp.int32),
        pltpu.VMEM((LMAX,), jnp.int32),
        pltpu.SemaphoreType.DMA,
        pltpu.SemaphoreType.DMA,
        pltpu.SemaphoreType.DMA,
        pltpu.SemaphoreType.DMA,
        pltpu.SemaphoreType.DMA,
    ],
)
def _triu_sc(x_ref, idx_ref, out_ref, slab0, slab1, buf0, buf1, idxA, idxB,
             ss0, ss1, ts0, ts1, is0):
    wid = lax.axis_index("s") * NC + lax.axis_index("c")
    base = wid * BATCH_PER_W
    idxbufs = (idxA, idxB)

    def idx_cp(g, buf):
        return pltpu.make_async_copy(
            idx_ref.at[pl.ds(POFF[g], PLEN[g])], buf.at[pl.ds(0, PLEN[g])], is0)

    def slab_cp(g, b, sl, sem):
        w = GWID[g]
        return pltpu.make_async_copy(
            x_ref.at[b, pl.ds(g * GROUP_ROWS, GROUP_ROWS), pl.ds(g * GROUP_ROWS, w)],
            sl.at[:, pl.ds(0, w)], sem)

    idx_cp(0, idxA).start()
    idx_cp(0, idxA).wait()
    slab_cp(0, base, slab0, ss0).start()

    for g in range(NGROUPS):
        goff, glen, plen = GOFF[g], GLEN[g], PLEN[g]
        idxg = idxbufs[g % 2]

        def out_cp(b, buf, sem, goff=goff, glen=glen):
            return pltpu.make_async_copy(
                buf.at[pl.ds(0, glen)],
                out_ref.at[pl.ds(b * OUT_LEN + goff, glen)], sem)

        def gather(slab, buf, plen=plen, idxg=idxg):
            @plsc.parallel_loop(0, plen, 64, unroll=4)
            def _(o):
                for k in range(4):
                    ids = idxg[pl.ds(o + k * 16, 16)]
                    rows = lax.shift_right_logical(ids, 9)
                    cols = lax.bitwise_and(ids, N - 1)
                    buf[pl.ds(o + k * 16, 16)] = plsc.load_gather(
                        slab, [rows, cols])

        def pair_body(i2, _, g=g):
            i = i2 * 2
            ba = base + i
            slab_cp(g, ba, slab0, ss0).wait()
            slab_cp(g, ba + 1, slab1, ss1).start()

            @pl.when(i > 0)
            def _():
                out_cp(ba - 2, buf0, ts0).wait()

            gather(slab0, buf0)
            out_cp(ba, buf0, ts0).start()

            slab_cp(g, ba + 1, slab1, ss1).wait()

            @pl.when(i < BATCH_PER_W - 2)
            def _():
                slab_cp(g, ba + 2, slab0, ss0).start()

            @pl.when(i > 0)
            def _():
                out_cp(ba - 1, buf1, ts1).wait()

            gather(slab1, buf1)
            out_cp(ba + 1, buf1, ts1).start()
            return 0

        lax.fori_loop(0, BATCH_PER_W // 2, pair_body, 0, unroll=False)

        if g + 1 < NGROUPS:
            idx_cp(g + 1, idxbufs[(g + 1) % 2]).start()
            slab_cp(g + 1, base, slab0, ss0).start()
            idx_cp(g + 1, idxbufs[(g + 1) % 2]).wait()

        out_cp(base + BATCH_PER_W - 2, buf0, ts0).wait()
        out_cp(base + BATCH_PER_W - 1, buf1, ts1).wait()


def kernel(x):
    out = _triu_sc(x, jnp.asarray(IDX_LOCAL))
    return out.reshape(B, OUT_LEN)


# speedup vs baseline: 2.8935x; 1.5832x over previous
import functools

import numpy as np
import jax
import jax.numpy as jnp
from jax import lax
from jax.experimental import pallas as pl
from jax.experimental.pallas import tpu as pltpu
from jax.experimental.pallas import tpu_sc as plsc

B = 256
N = 512
OUT_LEN = N * (N + 1) // 2
GROUP_ROWS = 32
NGROUPS = N // GROUP_ROWS

NC, NS = 2, 16
NW = NC * NS
BATCH_PER_W = B // NW

_r, _c = np.triu_indices(N)
_g = _r // GROUP_ROWS

GLEN = [int(np.sum(_g == g)) for g in range(NGROUPS)]
GOFF = [int(np.searchsorted(_g, g)) for g in range(NGROUPS)]
PLEN = [-(-l // 64) * 64 for l in GLEN]
POFF = list(np.cumsum([0] + PLEN[:-1]))
LMAX = PLEN[0]
GWID = [N - GROUP_ROWS * g for g in range(NGROUPS)]

_local = (_r - _g * GROUP_ROWS) * N + (_c - _g * GROUP_ROWS)
_idx_parts = []
for _gg in range(NGROUPS):
    _part = _local[GOFF[_gg]:GOFF[_gg] + GLEN[_gg]]
    _idx_parts.append(np.pad(_part, (0, PLEN[_gg] - GLEN[_gg])))
IDX_LOCAL = np.concatenate(_idx_parts).astype(np.int32)


@functools.partial(
    pl.kernel,
    mesh=plsc.VectorSubcoreMesh(core_axis_name="c", subcore_axis_name="s"),
    out_type=jax.ShapeDtypeStruct((B * OUT_LEN,), jnp.float32),
    compiler_params=pltpu.CompilerParams(
        needs_layout_passes=False, use_tc_tiling_on_sc=False),
    scratch_types=[
        pltpu.VMEM((GROUP_ROWS, N), jnp.float32),
        pltpu.VMEM((GROUP_ROWS, N), jnp.float32),
        pltpu.VMEM((LMAX,), jnp.float32),
        pltpu.VMEM((LMAX,), jnp.float32),
        pltpu.VMEM((LMAX,), jnp.int32),
        pltpu.VMEM((LMAX,), jnp.int32),
        pltpu.SemaphoreType.DMA,
        pltpu.SemaphoreType.DMA,
        pltpu.SemaphoreType.DMA,
        pltpu.SemaphoreType.DMA,
        pltpu.SemaphoreType.DMA,
    ],
)
def _triu_sc(x_ref, idx_ref, out_ref, slab0, slab1, buf0, buf1, idxA, idxB,
             ss0, ss1, ts0, ts1, is0):
    wid = lax.axis_index("s") * NC + lax.axis_index("c")
    base = wid * BATCH_PER_W
    idxbufs = (idxA, idxB)

    def idx_cp(g, buf):
        return pltpu.make_async_copy(
            idx_ref.at[pl.ds(POFF[g], PLEN[g])], buf.at[pl.ds(0, PLEN[g])], is0)

    def slab_cp(g, b, sl, sem):
        w = GWID[g]
        return pltpu.make_async_copy(
            x_ref.at[b, pl.ds(g * GROUP_ROWS, GROUP_ROWS), pl.ds(g * GROUP_ROWS, w)],
            sl.at[:, pl.ds(0, w)], sem)

    idx_cp(0, idxA).start()
    idx_cp(0, idxA).wait()
    slab_cp(0, base, slab0, ss0).start()

    for g in range(NGROUPS):
        goff, glen, plen = GOFF[g], GLEN[g], PLEN[g]
        idxg = idxbufs[g % 2]

        def out_cp(b, buf, sem, goff=goff, glen=glen):
            return pltpu.make_async_copy(
                buf.at[pl.ds(0, glen)],
                out_ref.at[pl.ds(b * OUT_LEN + goff, glen)], sem)

        def gather(slab, buf, plen=plen, idxg=idxg):
            @plsc.parallel_loop(0, plen, 64, unroll=4)
            def _(o):
                for k in range(4):
                    ids = idxg[pl.ds(o + k * 16, 16)]
                    rows = lax.shift_right_logical(ids, 9)
                    cols = lax.bitwise_and(ids, N - 1)
                    buf[pl.ds(o + k * 16, 16)] = plsc.load_gather(
                        slab, [rows, cols])

        def pair_body(i2, _, g=g):
            i = i2 * 2
            ba = base + i
            slab_cp(g, ba, slab0, ss0).wait()
            slab_cp(g, ba + 1, slab1, ss1).start()

            @pl.when(i > 0)
            def _():
                out_cp(ba - 2, buf0, ts0).wait()

            gather(slab0, buf0)
            out_cp(ba, buf0, ts0).start()

            slab_cp(g, ba + 1, slab1, ss1).wait()

            @pl.when(i < BATCH_PER_W - 2)
            def _():
                slab_cp(g, ba + 2, slab0, ss0).start()

            @pl.when(i > 0)
            def _():
                out_cp(ba - 1, buf1, ts1).wait()

            gather(slab1, buf1)
            out_cp(ba + 1, buf1, ts1).start()
            return 0

        lax.fori_loop(0, BATCH_PER_W // 2, pair_body, 0, unroll=False)

        if g + 1 < NGROUPS:
            idx_cp(g + 1, idxbufs[(g + 1) % 2]).start()
            slab_cp(g + 1, base, slab0, ss0).start()
            idx_cp(g + 1, idxbufs[(g + 1) % 2]).wait()

        out_cp(base + BATCH_PER_W - 2, buf0, ts0).wait()
        out_cp(base + BATCH_PER_W - 1, buf1, ts1).wait()


_OFFR = [r * N - r * (r - 1) // 2 for r in range(N + 1)]


def _tc_body(x_ref, out_ref):
    for r in range(N - 1, -1, -1):
        out_ref[0, 0, pl.ds(_OFFR[r] - r, N)] = x_ref[0, r, :]


def _triu_tc(xs):
    nb = xs.shape[0]
    out = pl.pallas_call(
        _tc_body,
        grid=(nb,),
        in_specs=[pl.BlockSpec((1, N, N), lambda b: (b, 0, 0))],
        out_specs=pl.BlockSpec((1, 1, OUT_LEN), lambda b: (b, 0, 0)),
        out_shape=jax.ShapeDtypeStruct((nb, 1, OUT_LEN), jnp.float32),
    )(xs)
    return out.reshape(nb, OUT_LEN)


def kernel(x):
    return _triu_tc(x)
